# baseline (speedup 1.0000x reference)
import jax
import jax.numpy as jnp
from jax import lax
from jax.experimental import pallas as pl
from jax.experimental.pallas import tpu as pltpu
from jax.experimental.pallas import tpu_sc as plsc

NUM_CORES = 2
NUM_SUBCORES = 16
NUM_WORKERS = NUM_CORES * NUM_SUBCORES

B = 4096
S = 200
D = 64
V = 1000000
N = B * S

CHUNK = 128
CHUNKS_PER_W = N // (NUM_WORKERS * CHUNK)
NBUF = 4
LOOKAHEAD = 2

TBLK = 1024


NBLK = (V + TBLK - 1) // TBLK
VPAD = NBLK * TBLK


def _relayout_body(src, dst):
    y = src[...].T
    dst[:, 0:D] = y[0:TBLK // 2]
    dst[:, D:2 * D] = y[TBLK // 2:TBLK]


@jax.jit
def _tc_table_rows(tab_t):
    return pl.pallas_call(
        _relayout_body,
        grid=(NBLK,),
        in_specs=[pl.BlockSpec((D, TBLK), lambda i: (0, i))],
        out_specs=pl.BlockSpec((TBLK // 2, 128), lambda i: (i, 0)),
        out_shape=jax.ShapeDtypeStruct((VPAD // 2, 128), jnp.float32),
    )(tab_t)


BB = 1024


def _out_relayout_body(src, dst):
    dst[0] = src[0].T


@jax.jit
def _tc_out_sdb(g3):
    return pl.pallas_call(
        _out_relayout_body,
        grid=(S, B // BB),
        in_specs=[pl.BlockSpec((1, BB, D), lambda i, j: (i, j, 0))],
        out_specs=pl.BlockSpec((1, D, BB), lambda i, j: (i, 0, j)),
        out_shape=jax.ShapeDtypeStruct((S, D, B), jnp.float32),
    )(g3)


@jax.jit
def _sc_gather(table, idx2):
    def body(tab, idxr, out, idx_v, rows, gsem, osem):
        c = lax.axis_index("c")
        sc = lax.axis_index("s")
        w = sc * NUM_CORES + c
        row0 = w * CHUNKS_PER_W
        base = row0 * CHUNK

        pltpu.sync_copy(idxr.at[pl.ds(row0, CHUNKS_PER_W)], idx_v)

        def start_gather(k, b):
            pltpu.make_async_copy(
                tab.at[idx_v.at[k]], rows.at[b], gsem.at[b]
            ).start()

        def wait_gather(k, b):
            pltpu.make_async_copy(
                tab.at[idx_v.at[k]], rows.at[b], gsem.at[b]
            ).wait()

        def start_out(k, b):
            pltpu.make_async_copy(
                rows.at[b], out.at[pl.ds(base + k * CHUNK, CHUNK)], osem.at[b]
            ).start()

        def wait_out(k, b):
            pltpu.make_async_copy(
                rows.at[b], out.at[pl.ds(base + k * CHUNK, CHUNK)], osem.at[b]
            ).wait()

        for k in range(LOOKAHEAD):
            start_gather(k, k % NBUF)

        def step(i, carry):
            for bb in range(NBUF):
                k = i * NBUF + bb
                wait_gather(k, bb)
                start_out(k, bb)

                nk = k + LOOKAHEAD
                nb = (bb + LOOKAHEAD) % NBUF

                @pl.when(nk < CHUNKS_PER_W)
                def _():
                    @pl.when(nk >= NBUF)
                    def _():
                        wait_out(nk - NBUF, nb)

                    start_gather(nk, nb)

            return carry

        lax.fori_loop(0, CHUNKS_PER_W // NBUF, step, 0)

        for k in range(CHUNKS_PER_W - NBUF, CHUNKS_PER_W):
            wait_out(k, k % NBUF)

    run = pl.kernel(
        body,
        out_type=jax.ShapeDtypeStruct((N, D), jnp.float32),
        mesh=plsc.VectorSubcoreMesh(core_axis_name="c", subcore_axis_name="s"),
        scratch_types=[
            pltpu.VMEM((CHUNKS_PER_W, CHUNK), jnp.int32),
            pltpu.VMEM((NBUF, CHUNK, D), jnp.float32),
            pltpu.SemaphoreType.DMA((NBUF,)),
            pltpu.SemaphoreType.DMA((NBUF,)),
        ],
        compiler_params=pltpu.CompilerParams(use_tc_tiling_on_sc=False),
    )
    return run(table, idx2)


def kernel(indices, table):
    tab2 = _tc_table_rows(table.T)
    tbl_lin = tab2.reshape(VPAD, D)
    vi = indices.astype(jnp.int32)
    u = (vi & ~(TBLK - 1)) + ((vi & (TBLK // 2 - 1)) << 1) + (
        (vi >> 9) & 1
    )
    idx2 = u.T.reshape(N // CHUNK, CHUNK)
    gout = _sc_gather(tbl_lin, idx2)
    g3 = gout.reshape(S, B, D)
    out_sdb = _tc_out_sdb(g3)
    return out_sdb.transpose(2, 0, 1)


# speedup vs baseline: 1.2818x; 1.2818x over previous
import jax
import jax.numpy as jnp
from jax import lax
from jax.experimental import pallas as pl
from jax.experimental.pallas import tpu as pltpu
from jax.experimental.pallas import tpu_sc as plsc

NUM_CORES = 2
NUM_SUBCORES = 16
NUM_WORKERS = NUM_CORES * NUM_SUBCORES

B = 4096
S = 200
D = 64
V = 1000000
N = B * S

CHUNK = 128
CHUNKS_PER_W = N // (NUM_WORKERS * CHUNK)
NBUF = 4
LOOKAHEAD = 2

TBLK = 1024


NBLK = (V + TBLK - 1) // TBLK
VPAD = NBLK * TBLK


def _relayout_body(src, dst):
    y = src[...].T
    dst[:, 0:D] = y[0:TBLK // 2]
    dst[:, D:2 * D] = y[TBLK // 2:TBLK]


@jax.jit
def _tc_table_rows(tab_t):
    return pl.pallas_call(
        _relayout_body,
        grid=(NBLK,),
        in_specs=[pl.BlockSpec((D, TBLK), lambda i: (0, i))],
        out_specs=pl.BlockSpec((TBLK // 2, 128), lambda i: (i, 0)),
        out_shape=jax.ShapeDtypeStruct((VPAD // 2, 128), jnp.float32),
    )(tab_t)


H = B // 2


def _out_relayout_body(src, dst):
    x = src[0]
    dst[0, :, 0:H] = x[:, 0:D].T
    dst[0, :, H:B] = x[:, D:2 * D].T


@jax.jit
def _tc_out_sdb(g3):
    return pl.pallas_call(
        _out_relayout_body,
        grid=(S,),
        in_specs=[pl.BlockSpec((1, H, 128), lambda i: (i, 0, 0))],
        out_specs=pl.BlockSpec((1, D, B), lambda i: (i, 0, 0)),
        out_shape=jax.ShapeDtypeStruct((S, D, B), jnp.float32),
    )(g3)


@jax.jit
def _sc_gather(table, idx2):
    def body(tab, idxr, out, idx_v, rows, gsem, osem):
        c = lax.axis_index("c")
        sc = lax.axis_index("s")
        w = sc * NUM_CORES + c
        row0 = w * CHUNKS_PER_W
        base = row0 * CHUNK

        pltpu.sync_copy(idxr.at[pl.ds(row0, CHUNKS_PER_W)], idx_v)

        def start_gather(k, b):
            pltpu.make_async_copy(
                tab.at[idx_v.at[k]], rows.at[b], gsem.at[b]
            ).start()

        def wait_gather(k, b):
            pltpu.make_async_copy(
                tab.at[idx_v.at[k]], rows.at[b], gsem.at[b]
            ).wait()

        def start_out(k, b):
            pltpu.make_async_copy(
                rows.at[b], out.at[pl.ds(base + k * CHUNK, CHUNK)], osem.at[b]
            ).start()

        def wait_out(k, b):
            pltpu.make_async_copy(
                rows.at[b], out.at[pl.ds(base + k * CHUNK, CHUNK)], osem.at[b]
            ).wait()

        for k in range(LOOKAHEAD):
            start_gather(k, k % NBUF)

        def step(i, carry):
            for bb in range(NBUF):
                k = i * NBUF + bb
                wait_gather(k, bb)
                start_out(k, bb)

                nk = k + LOOKAHEAD
                nb = (bb + LOOKAHEAD) % NBUF

                @pl.when(nk < CHUNKS_PER_W)
                def _():
                    @pl.when(nk >= NBUF)
                    def _():
                        wait_out(nk - NBUF, nb)

                    start_gather(nk, nb)

            return carry

        lax.fori_loop(0, CHUNKS_PER_W // NBUF, step, 0)

        for k in range(CHUNKS_PER_W - NBUF, CHUNKS_PER_W):
            wait_out(k, k % NBUF)

    run = pl.kernel(
        body,
        out_type=jax.ShapeDtypeStruct((N, D), jnp.float32),
        mesh=plsc.VectorSubcoreMesh(core_axis_name="c", subcore_axis_name="s"),
        scratch_types=[
            pltpu.VMEM((CHUNKS_PER_W, CHUNK), jnp.int32),
            pltpu.VMEM((NBUF, CHUNK, D), jnp.float32),
            pltpu.SemaphoreType.DMA((NBUF,)),
            pltpu.SemaphoreType.DMA((NBUF,)),
        ],
        compiler_params=pltpu.CompilerParams(use_tc_tiling_on_sc=False),
    )
    return run(table, idx2)


def kernel(indices, table):
    tab2 = _tc_table_rows(table.T)
    tbl_lin = tab2.reshape(VPAD, D)
    vi = indices.astype(jnp.int32)
    u = (vi & ~(TBLK - 1)) + ((vi & (TBLK // 2 - 1)) << 1) + (
        (vi >> 9) & 1
    )
    ut = u.T.reshape(S, 2, H).transpose(0, 2, 1)
    idx2 = ut.reshape(N // CHUNK, CHUNK)
    gout = _sc_gather(tbl_lin, idx2)
    g3 = gout.reshape(S, H, 128)
    out_sdb = _tc_out_sdb(g3)
    return out_sdb.transpose(2, 0, 1)


# speedup vs baseline: 1.7474x; 1.3633x over previous
import jax
import jax.numpy as jnp
from jax import lax
from jax.experimental import pallas as pl
from jax.experimental.pallas import tpu as pltpu
from jax.experimental.pallas import tpu_sc as plsc

NUM_CORES = 2
NUM_SUBCORES = 16
NUM_WORKERS = NUM_CORES * NUM_SUBCORES

B = 4096
S = 200
D = 64
V = 1000000
N = B * S

CHUNK = 128
CHUNKS_PER_W = N // (NUM_WORKERS * CHUNK)
NBUF = 4
LOOKAHEAD = 2

TBLK = 4096


NBLK = (V + TBLK - 1) // TBLK
VPAD = NBLK * TBLK


def _relayout_body(src, dst):
    y = src[...].T
    dst[:, 0:D] = y[0:TBLK // 2]
    dst[:, D:2 * D] = y[TBLK // 2:TBLK]


@jax.jit
def _tc_table_rows(tab_t):
    return pl.pallas_call(
        _relayout_body,
        grid=(NBLK,),
        in_specs=[pl.BlockSpec((D, TBLK), lambda i: (0, i))],
        out_specs=pl.BlockSpec((TBLK // 2, 128), lambda i: (i, 0)),
        out_shape=jax.ShapeDtypeStruct((VPAD // 2, 128), jnp.float32),
    )(tab_t)


H = B // 2


def _out_relayout_body(src, dst):
    x = src[0]
    dst[0, :, 0:H] = x[:, 0:D].T
    dst[0, :, H:B] = x[:, D:2 * D].T


@jax.jit
def _tc_out_sdb(g3):
    return pl.pallas_call(
        _out_relayout_body,
        grid=(S,),
        in_specs=[pl.BlockSpec((1, H, 128), lambda i: (i, 0, 0))],
        out_specs=pl.BlockSpec((1, D, B), lambda i: (i, 0, 0)),
        out_shape=jax.ShapeDtypeStruct((S, D, B), jnp.float32),
    )(g3)


@jax.jit
def _sc_gather(table, idx2):
    def body(tab, idxr, out, idx_v, rows, gsem, osem):
        c = lax.axis_index("c")
        sc = lax.axis_index("s")
        w = sc * NUM_CORES + c
        row0 = w * CHUNKS_PER_W
        base = row0 * CHUNK

        pltpu.sync_copy(idxr.at[pl.ds(row0, CHUNKS_PER_W)], idx_v)

        def start_gather(k, b):
            pltpu.make_async_copy(
                tab.at[idx_v.at[k]], rows.at[b], gsem.at[b]
            ).start()

        def wait_gather(k, b):
            pltpu.make_async_copy(
                tab.at[idx_v.at[k]], rows.at[b], gsem.at[b]
            ).wait()

        def start_out(k, b):
            pltpu.make_async_copy(
                rows.at[b], out.at[pl.ds(base + k * CHUNK, CHUNK)], osem.at[b]
            ).start()

        def wait_out(k, b):
            pltpu.make_async_copy(
                rows.at[b], out.at[pl.ds(base + k * CHUNK, CHUNK)], osem.at[b]
            ).wait()

        for k in range(LOOKAHEAD):
            start_gather(k, k % NBUF)

        def step(i, carry):
            for bb in range(NBUF):
                k = i * NBUF + bb
                wait_gather(k, bb)
                start_out(k, bb)

                nk = k + LOOKAHEAD
                nb = (bb + LOOKAHEAD) % NBUF

                @pl.when(nk < CHUNKS_PER_W)
                def _():
                    @pl.when(nk >= NBUF)
                    def _():
                        wait_out(nk - NBUF, nb)

                    start_gather(nk, nb)

            return carry

        lax.fori_loop(0, CHUNKS_PER_W // NBUF, step, 0)

        for k in range(CHUNKS_PER_W - NBUF, CHUNKS_PER_W):
            wait_out(k, k % NBUF)

    run = pl.kernel(
        body,
        out_type=jax.ShapeDtypeStruct((N, D), jnp.float32),
        mesh=plsc.VectorSubcoreMesh(core_axis_name="c", subcore_axis_name="s"),
        scratch_types=[
            pltpu.VMEM((CHUNKS_PER_W, CHUNK), jnp.int32),
            pltpu.VMEM((NBUF, CHUNK, D), jnp.float32),
            pltpu.SemaphoreType.DMA((NBUF,)),
            pltpu.SemaphoreType.DMA((NBUF,)),
        ],
        compiler_params=pltpu.CompilerParams(use_tc_tiling_on_sc=False),
    )
    return run(table, idx2)


def kernel(indices, table):
    tab2 = _tc_table_rows(table.T)
    tbl_lin = tab2.reshape(VPAD, D)
    vi = indices.astype(jnp.int32)
    u = (vi & ~(TBLK - 1)) + ((vi & (TBLK // 2 - 1)) << 1) + (
        (vi >> (TBLK // 2).bit_length() - 1) & 1
    )
    ut = u.T.reshape(S, 2, H).transpose(0, 2, 1)
    idx2 = ut.reshape(N // CHUNK, CHUNK)
    gout = _sc_gather(tbl_lin, idx2)
    g3 = gout.reshape(S, H, 128)
    out_sdb = _tc_out_sdb(g3)
    return out_sdb.transpose(2, 0, 1)


# speedup vs baseline: 1.8722x; 1.0714x over previous
import jax
import jax.numpy as jnp
from jax import lax
from jax.experimental import pallas as pl
from jax.experimental.pallas import tpu as pltpu
from jax.experimental.pallas import tpu_sc as plsc

NUM_CORES = 2
NUM_SUBCORES = 16
NUM_WORKERS = NUM_CORES * NUM_SUBCORES

B = 4096
S = 200
D = 64
V = 1000000
N = B * S

CHUNK = 128
CHUNKS_PER_W = N // (NUM_WORKERS * CHUNK)
NBUF = 4
LOOKAHEAD = 2

TBLK = 8192


NBLK = (V + TBLK - 1) // TBLK
VPAD = NBLK * TBLK


def _relayout_body(src, dst):
    y = src[...].T
    dst[:, 0:D] = y[0:TBLK // 2]
    dst[:, D:2 * D] = y[TBLK // 2:TBLK]


@jax.jit
def _tc_table_rows(tab_t):
    return pl.pallas_call(
        _relayout_body,
        grid=(NBLK,),
        in_specs=[pl.BlockSpec((D, TBLK), lambda i: (0, i))],
        out_specs=pl.BlockSpec((TBLK // 2, 128), lambda i: (i, 0)),
        out_shape=jax.ShapeDtypeStruct((VPAD // 2, 128), jnp.float32),
    )(tab_t)


H = B // 2


def _out_relayout_body(src, dst):
    x = src[0]
    dst[0, :, 0:H] = x[:, 0:D].T
    dst[0, :, H:B] = x[:, D:2 * D].T


@jax.jit
def _tc_out_sdb(g3):
    return pl.pallas_call(
        _out_relayout_body,
        grid=(S,),
        in_specs=[pl.BlockSpec((1, H, 128), lambda i: (i, 0, 0))],
        out_specs=pl.BlockSpec((1, D, B), lambda i: (i, 0, 0)),
        out_shape=jax.ShapeDtypeStruct((S, D, B), jnp.float32),
    )(g3)


@jax.jit
def _sc_gather(table, idx2):
    def body(tab, idxr, out, idx_v, rows, gsem, osem):
        c = lax.axis_index("c")
        sc = lax.axis_index("s")
        w = sc * NUM_CORES + c
        row0 = w * CHUNKS_PER_W
        base = row0 * CHUNK

        pltpu.sync_copy(idxr.at[pl.ds(row0, CHUNKS_PER_W)], idx_v)

        def start_gather(k, b):
            pltpu.make_async_copy(
                tab.at[idx_v.at[k]], rows.at[b], gsem.at[b]
            ).start()

        def wait_gather(k, b):
            pltpu.make_async_copy(
                tab.at[idx_v.at[k]], rows.at[b], gsem.at[b]
            ).wait()

        def start_out(k, b):
            pltpu.make_async_copy(
                rows.at[b], out.at[pl.ds(base + k * CHUNK, CHUNK)], osem.at[b]
            ).start()

        def wait_out(k, b):
            pltpu.make_async_copy(
                rows.at[b], out.at[pl.ds(base + k * CHUNK, CHUNK)], osem.at[b]
            ).wait()

        for k in range(LOOKAHEAD):
            start_gather(k, k % NBUF)

        def step(i, carry):
            for bb in range(NBUF):
                k = i * NBUF + bb
                wait_gather(k, bb)
                start_out(k, bb)

                nk = k + LOOKAHEAD
                nb = (bb + LOOKAHEAD) % NBUF

                @pl.when(nk < CHUNKS_PER_W)
                def _():
                    @pl.when(nk >= NBUF)
                    def _():
                        wait_out(nk - NBUF, nb)

                    start_gather(nk, nb)

            return carry

        lax.fori_loop(0, CHUNKS_PER_W // NBUF, step, 0)

        for k in range(CHUNKS_PER_W - NBUF, CHUNKS_PER_W):
            wait_out(k, k % NBUF)

    run = pl.kernel(
        body,
        out_type=jax.ShapeDtypeStruct((N, D), jnp.float32),
        mesh=plsc.VectorSubcoreMesh(core_axis_name="c", subcore_axis_name="s"),
        scratch_types=[
            pltpu.VMEM((CHUNKS_PER_W, CHUNK), jnp.int32),
            pltpu.VMEM((NBUF, CHUNK, D), jnp.float32),
            pltpu.SemaphoreType.DMA((NBUF,)),
            pltpu.SemaphoreType.DMA((NBUF,)),
        ],
        compiler_params=pltpu.CompilerParams(use_tc_tiling_on_sc=False),
    )
    return run(table, idx2)


def kernel(indices, table):
    tab2 = _tc_table_rows(table.T)
    tbl_lin = tab2.reshape(VPAD, D)
    vi = indices.astype(jnp.int32)
    u = (vi & ~(TBLK - 1)) + ((vi & (TBLK // 2 - 1)) << 1) + (
        (vi >> (TBLK // 2).bit_length() - 1) & 1
    )
    ut = u.T.reshape(S, 2, H).transpose(0, 2, 1)
    idx2 = ut.reshape(N // CHUNK, CHUNK)
    gout = _sc_gather(tbl_lin, idx2)
    g3 = gout.reshape(S, H, 128)
    out_sdb = _tc_out_sdb(g3)
    return out_sdb.transpose(2, 0, 1)
